# baseline (speedup 1.0000x reference)
import functools

import jax
import jax.numpy as jnp
from jax import lax
from jax.experimental import pallas as pl
from jax.experimental.pallas import tpu as pltpu
from jax.experimental.pallas import tpu_sc as plsc

N = 10000
E = 320000
D = 128
H = 128
C = 47
CP = 128

NC = 2
NS = 16
NT = NC * NS
K = 80
NCHUNK = 128
E2 = NT * NCHUNK * K
CH0 = 156
CH1 = 2 * NCHUNK - CH0
NP = 10240
RPT = NP // NS
RB = K
NB = RPT // RB


def _mm_body(h_ref, w_ref, o_ref):
    o_ref[...] = jnp.dot(h_ref[...], w_ref[...],
                         precision=lax.Precision.HIGHEST,
                         preferred_element_type=jnp.float32)


def _matmul(h, w):
    n, d = h.shape
    m = w.shape[1]
    bn = 512
    return pl.pallas_call(
        _mm_body,
        grid=(pl.cdiv(n, bn),),
        in_specs=[pl.BlockSpec((bn, d), lambda i: (i, 0)),
                  pl.BlockSpec((d, m), lambda i: (0, 0))],
        out_specs=pl.BlockSpec((bn, m), lambda i: (i, 0)),
        out_shape=jax.ShapeDtypeStruct((n, m), jnp.float32),
    )(h, w)


def _combine_body(relu, h_ref, w_ref, b_ref, a0_ref, a1_ref, dp_ref, o_ref):
    deg = jnp.sum(dp_ref[...], axis=0)[:, None]
    inv = 1.0 / jnp.maximum(deg, 1.0)
    o = (jnp.dot(h_ref[...], w_ref[...],
                 precision=lax.Precision.HIGHEST,
                 preferred_element_type=jnp.float32)
         + b_ref[...] + (a0_ref[...] + a1_ref[...]) * inv)
    if relu:
        o = jnp.maximum(o, 0.0)
    o_ref[...] = o


def _combine(h, w, b, a0, a1, dp, relu):
    n, d = h.shape
    m = w.shape[1]
    bn = 512
    return pl.pallas_call(
        functools.partial(_combine_body, relu),
        grid=(pl.cdiv(n, bn),),
        in_specs=[pl.BlockSpec((bn, d), lambda i: (i, 0)),
                  pl.BlockSpec((d, m), lambda i: (0, 0)),
                  pl.BlockSpec((1, m), lambda i: (0, 0)),
                  pl.BlockSpec((bn, m), lambda i: (i, 0)),
                  pl.BlockSpec((bn, m), lambda i: (i, 0)),
                  pl.BlockSpec((NT, bn), lambda i: (0, i))],
        out_specs=pl.BlockSpec((bn, m), lambda i: (i, 0)),
        out_shape=jax.ShapeDtypeStruct((n, m), jnp.float32),
    )(h, w, b, a0, a1, dp)


def _make_sc_agg(w, with_deg):
    mesh = plsc.VectorSubcoreMesh(core_axis_name="c", subcore_axis_name="s")
    out_type = [jax.ShapeDtypeStruct((NC, NP, w), jnp.float32)]
    scratch = [
        pltpu.VMEM_SHARED((NP, w), jnp.float32),
        pltpu.VMEM((K,), jnp.int32),
        pltpu.VMEM((K,), jnp.int32),
        pltpu.VMEM((K,), jnp.int32),
        pltpu.VMEM((K,), jnp.int32),
        pltpu.VMEM((K, w), jnp.float32),
        pltpu.VMEM((K, w), jnp.float32),
        pltpu.VMEM((RB,), jnp.int32),
        pltpu.VMEM((RB,), jnp.int32),
        pltpu.SemaphoreType.DMA,
        pltpu.SemaphoreType.DMA,
    ]
    if with_deg:
        out_type.append(jax.ShapeDtypeStruct((NT, NP), jnp.float32))
        scratch.append(pltpu.VMEM((NP,), jnp.float32))

    @functools.partial(
        pl.kernel, out_type=out_type, mesh=mesh, scratch_types=scratch,
        compiler_params=pltpu.CompilerParams(needs_layout_passes=False))
    def sc_agg(*refs):
        if with_deg:
            (t_hbm, src_hbm, dst_hbm, z_hbm, zdeg_hbm,
             out_acc, out_deg,
             acc_sh, src0_v, src1_v, dst0_v, dst1_v, rows0_v, rows1_v,
             ix0_v, ix1_v, sem0, sem1, deg_v) = refs
        else:
            (t_hbm, src_hbm, dst_hbm, z_hbm,
             out_acc,
             acc_sh, src0_v, src1_v, dst0_v, dst1_v, rows0_v, rows1_v,
             ix0_v, ix1_v, sem0, sem1) = refs
        cid = lax.axis_index("c")
        sid = lax.axis_index("s")
        wid = sid * NC + cid
        row0 = sid * RPT
        iota = lax.iota(jnp.int32, 16)
        ones_lane = jnp.ones((16,), jnp.float32)
        pltpu.sync_copy(z_hbm, rows0_v)
        if with_deg:
            pltpu.sync_copy(zdeg_hbm, deg_v)

        def fill_ix(ix_v, r):
            for ii in range(RB // 16):
                ix_v[pl.ds(ii * 16, 16)] = iota + (r + ii * 16)

        def zero_chunk(i, carry):
            fill_ix(ix0_v, row0 + i * RB)
            pltpu.sync_copy(rows0_v, acc_sh.at[ix0_v])
            return carry

        lax.fori_loop(0, NB, zero_chunk, 0)
        plsc.subcore_barrier()

        def scatter_chunk(rows_v, dst_v):
            pltpu.sync_copy(rows_v, acc_sh.at[dst_v], add=True)
            if with_deg:
                for ii in range(K // 16):
                    idx = dst_v[pl.ds(ii * 16, 16)]
                    plsc.addupdate_scatter(deg_v, [idx], ones_lane)

        def load_idx(base, src_v, dst_v, c):
            off = pl.multiple_of(base + c * K, 8)
            pltpu.sync_copy(src_hbm.at[pl.ds(off, K)], src_v)
            pltpu.sync_copy(dst_hbm.at[pl.ds(off, K)], dst_v)

        @pl.when(cid == 0)
        def _():
            base = sid * CH0 * K
            load_idx(base, src0_v, dst0_v, 0)
            pltpu.async_copy(t_hbm.at[src0_v], rows0_v, sem0)

            def pair(j, carry):
                c1 = 2 * j + 1
                c2 = jnp.minimum(2 * j + 2, CH0 - 1)
                load_idx(base, src1_v, dst1_v, c1)
                pltpu.make_async_copy(t_hbm.at[src0_v], rows0_v, sem0).wait()
                pltpu.async_copy(t_hbm.at[src1_v], rows1_v, sem1)
                scatter_chunk(rows0_v, dst0_v)
                load_idx(base, src0_v, dst0_v, c2)
                pltpu.make_async_copy(t_hbm.at[src1_v], rows1_v, sem1).wait()
                pltpu.async_copy(t_hbm.at[src0_v], rows0_v, sem0)
                scatter_chunk(rows1_v, dst1_v)
                return carry

            lax.fori_loop(0, CH0 // 2, pair, 0)
            pltpu.make_async_copy(t_hbm.at[src0_v], rows0_v, sem0).wait()

        @pl.when(cid == 1)
        def _():
            base = (NS * CH0 + sid * CH1) * K

            def one(c, carry):
                load_idx(base, src0_v, dst0_v, c)
                pltpu.async_copy(t_hbm.at[src0_v], rows0_v, sem0).wait()
                scatter_chunk(rows0_v, dst0_v)
                return carry

            lax.fori_loop(0, CH1, one, 0)

        plsc.subcore_barrier()

        fill_ix(ix0_v, row0)
        pltpu.async_copy(acc_sh.at[ix0_v], rows0_v, sem0)

        def read_pair(j, carry):
            c1 = 2 * j + 1
            c2 = jnp.minimum(2 * j + 2, NB - 1)
            fill_ix(ix1_v, row0 + c1 * RB)
            pltpu.make_async_copy(acc_sh.at[ix0_v], rows0_v, sem0).wait()
            pltpu.async_copy(acc_sh.at[ix1_v], rows1_v, sem1)
            r0 = pl.multiple_of(row0 + 2 * j * RB, 8)
            pltpu.sync_copy(rows0_v, out_acc.at[cid, pl.ds(r0, RB)])
            fill_ix(ix0_v, row0 + c2 * RB)
            pltpu.make_async_copy(acc_sh.at[ix1_v], rows1_v, sem1).wait()
            pltpu.async_copy(acc_sh.at[ix0_v], rows0_v, sem0)
            r1 = pl.multiple_of(row0 + c1 * RB, 8)
            pltpu.sync_copy(rows1_v, out_acc.at[cid, pl.ds(r1, RB)])
            return carry

        lax.fori_loop(0, NB // 2, read_pair, 0)
        pltpu.make_async_copy(acc_sh.at[ix0_v], rows0_v, sem0).wait()
        if with_deg:
            pltpu.sync_copy(deg_v, out_deg.at[wid])

    return sc_agg


_sc_agg_deg = _make_sc_agg(H, True)
_sc_agg_h = _make_sc_agg(H, False)


def kernel(x, edge_index, W_self0, W_neigh0, b0, W_self1, W_neigh1, b1,
           W_self2, W_neigh2, b2):
    src = jnp.concatenate([edge_index[0], jnp.zeros((E2 - E,), jnp.int32)])
    dst = jnp.concatenate(
        [edge_index[1], jnp.full((E2 - E,), NP - 1, jnp.int32)])
    zH = jnp.zeros((K, H), jnp.float32)
    zC = jnp.zeros((K, CP), jnp.float32)
    zdeg = jnp.zeros((NP,), jnp.float32)

    t0 = _matmul(x, W_neigh0)
    acc0, degp = _sc_agg_deg(t0, src, dst, zH, zdeg)
    dp = degp[:, :N]
    h1 = _combine(x, W_self0, b0.reshape(1, H), acc0[0, :N], acc0[1, :N],
                  dp, relu=True)

    t1 = _matmul(h1, W_neigh1)
    acc1 = _sc_agg_h(t1, src, dst, zH)[0]
    h2 = _combine(h1, W_self1, b1.reshape(1, H), acc1[0, :N], acc1[1, :N],
                  dp, relu=True)

    Wn2 = jnp.pad(W_neigh2, ((0, 0), (0, CP - C)))
    Ws2 = jnp.pad(W_self2, ((0, 0), (0, CP - C)))
    b2p = jnp.pad(b2, (0, CP - C)).reshape(1, CP)
    t2 = _matmul(h2, Wn2)
    acc2 = _sc_agg_h(t2, src, dst, zC)[0]
    out = _combine(h2, Ws2, b2p, acc2[0, :N], acc2[1, :N], dp, relu=False)
    return out[:, :C]


# speedup vs baseline: 1.1823x; 1.1823x over previous
import functools

import jax
import jax.numpy as jnp
from jax import lax
from jax.experimental import pallas as pl
from jax.experimental.pallas import tpu as pltpu
from jax.experimental.pallas import tpu_sc as plsc

N = 10000
E = 320000
D = 128
H = 128
C = 47
CP = 128

NC = 2
NS = 16
NT = NC * NS
EPT = 10240
E2 = NT * EPT
NP = 10240
RPT = NP // NS


def _mm_body(h_ref, w_ref, o_ref):
    o_ref[...] = jnp.dot(h_ref[...], w_ref[...],
                         precision=lax.Precision.HIGHEST,
                         preferred_element_type=jnp.float32)


def _matmul(h, w):
    n, d = h.shape
    m = w.shape[1]
    bn = 512
    return pl.pallas_call(
        _mm_body,
        grid=(pl.cdiv(n, bn),),
        in_specs=[pl.BlockSpec((bn, d), lambda i: (i, 0)),
                  pl.BlockSpec((d, m), lambda i: (0, 0))],
        out_specs=pl.BlockSpec((bn, m), lambda i: (i, 0)),
        out_shape=jax.ShapeDtypeStruct((n, m), jnp.float32),
    )(h, w)


def _combine_body(relu, h_ref, w_ref, b_ref, a0_ref, a1_ref, dp_ref, o_ref):
    deg = jnp.sum(dp_ref[...], axis=0)[:, None]
    inv = 1.0 / jnp.maximum(deg, 1.0)
    o = (jnp.dot(h_ref[...], w_ref[...],
                 precision=lax.Precision.HIGHEST,
                 preferred_element_type=jnp.float32)
         + b_ref[...] + (a0_ref[...] + a1_ref[...]) * inv)
    if relu:
        o = jnp.maximum(o, 0.0)
    o_ref[...] = o


def _combine(h, w, b, a0, a1, dp, relu):
    n, d = h.shape
    m = w.shape[1]
    bn = 512
    return pl.pallas_call(
        functools.partial(_combine_body, relu),
        grid=(pl.cdiv(n, bn),),
        in_specs=[pl.BlockSpec((bn, d), lambda i: (i, 0)),
                  pl.BlockSpec((d, m), lambda i: (0, 0)),
                  pl.BlockSpec((1, m), lambda i: (0, 0)),
                  pl.BlockSpec((bn, m), lambda i: (i, 0)),
                  pl.BlockSpec((bn, m), lambda i: (i, 0)),
                  pl.BlockSpec((NT, bn), lambda i: (0, i))],
        out_specs=pl.BlockSpec((bn, m), lambda i: (i, 0)),
        out_shape=jax.ShapeDtypeStruct((n, m), jnp.float32),
    )(h, w, b, a0, a1, dp)


def _make_sc_agg(w, with_deg, k):
    nchunk = EPT // k
    rb = k
    nb = RPT // rb
    mesh = plsc.VectorSubcoreMesh(core_axis_name="c", subcore_axis_name="s")
    out_type = [jax.ShapeDtypeStruct((NC, NP, w), jnp.float32)]
    scratch = [
        pltpu.VMEM_SHARED((NP, w), jnp.float32),
        pltpu.VMEM((k,), jnp.int32),
        pltpu.VMEM((k,), jnp.int32),
        pltpu.VMEM((k,), jnp.int32),
        pltpu.VMEM((k,), jnp.int32),
        pltpu.VMEM((k, w), jnp.float32),
        pltpu.VMEM((k, w), jnp.float32),
        pltpu.VMEM((rb,), jnp.int32),
        pltpu.VMEM((rb,), jnp.int32),
        pltpu.SemaphoreType.DMA,
        pltpu.SemaphoreType.DMA,
    ]
    if with_deg:
        out_type.append(jax.ShapeDtypeStruct((NT, NP), jnp.float32))
        scratch.append(pltpu.VMEM((NP,), jnp.float32))

    @functools.partial(
        pl.kernel, out_type=out_type, mesh=mesh, scratch_types=scratch,
        compiler_params=pltpu.CompilerParams(needs_layout_passes=False))
    def sc_agg(*refs):
        if with_deg:
            (t_hbm, src_hbm, dst_hbm, z_hbm, zdeg_hbm,
             out_acc, out_deg,
             acc_sh, src0_v, src1_v, dst0_v, dst1_v, rows0_v, rows1_v,
             ix0_v, ix1_v, sem0, sem1, deg_v) = refs
        else:
            (t_hbm, src_hbm, dst_hbm, z_hbm,
             out_acc,
             acc_sh, src0_v, src1_v, dst0_v, dst1_v, rows0_v, rows1_v,
             ix0_v, ix1_v, sem0, sem1) = refs
        cid = lax.axis_index("c")
        sid = lax.axis_index("s")
        wid = sid * NC + cid
        row0 = sid * RPT
        iota = lax.iota(jnp.int32, 16)
        ones_lane = jnp.ones((16,), jnp.float32)
        pltpu.sync_copy(z_hbm, rows0_v)
        if with_deg:
            pltpu.sync_copy(zdeg_hbm, deg_v)

        def fill_ix(ix_v, r):
            for ii in range(rb // 16):
                ix_v[pl.ds(ii * 16, 16)] = iota + (r + ii * 16)

        def zero_chunk(i, carry):
            fill_ix(ix0_v, row0 + i * rb)
            pltpu.sync_copy(rows0_v, acc_sh.at[ix0_v])
            return carry

        lax.fori_loop(0, nb, zero_chunk, 0)
        plsc.subcore_barrier()

        def scatter_chunk(rows_v, dst_v):
            pltpu.sync_copy(rows_v, acc_sh.at[dst_v], add=True)
            if with_deg:
                for ii in range(k // 16):
                    idx = dst_v[pl.ds(ii * 16, 16)]
                    plsc.addupdate_scatter(deg_v, [idx], ones_lane)

        base = wid * EPT

        def load_idx(src_v, dst_v, c):
            off = pl.multiple_of(base + c * k, 8)
            pltpu.sync_copy(src_hbm.at[pl.ds(off, k)], src_v)
            pltpu.sync_copy(dst_hbm.at[pl.ds(off, k)], dst_v)

        load_idx(src0_v, dst0_v, 0)
        pltpu.async_copy(t_hbm.at[src0_v], rows0_v, sem0)

        def pair(j, carry):
            c1 = 2 * j + 1
            c2 = jnp.minimum(2 * j + 2, nchunk - 1)
            load_idx(src1_v, dst1_v, c1)
            pltpu.make_async_copy(t_hbm.at[src0_v], rows0_v, sem0).wait()
            scatter_chunk(rows0_v, dst0_v)
            pltpu.async_copy(t_hbm.at[src1_v], rows1_v, sem1)
            load_idx(src0_v, dst0_v, c2)
            pltpu.make_async_copy(t_hbm.at[src1_v], rows1_v, sem1).wait()
            scatter_chunk(rows1_v, dst1_v)
            pltpu.async_copy(t_hbm.at[src0_v], rows0_v, sem0)
            return carry

        lax.fori_loop(0, nchunk // 2, pair, 0)
        pltpu.make_async_copy(t_hbm.at[src0_v], rows0_v, sem0).wait()
        plsc.subcore_barrier()

        fill_ix(ix0_v, row0)
        pltpu.async_copy(acc_sh.at[ix0_v], rows0_v, sem0)

        def read_pair(j, carry):
            c1 = 2 * j + 1
            c2 = jnp.minimum(2 * j + 2, nb - 1)
            fill_ix(ix1_v, row0 + c1 * rb)
            pltpu.make_async_copy(acc_sh.at[ix0_v], rows0_v, sem0).wait()
            pltpu.async_copy(acc_sh.at[ix1_v], rows1_v, sem1)
            r0 = pl.multiple_of(row0 + 2 * j * rb, 8)
            pltpu.sync_copy(rows0_v, out_acc.at[cid, pl.ds(r0, rb)])
            fill_ix(ix0_v, row0 + c2 * rb)
            pltpu.make_async_copy(acc_sh.at[ix1_v], rows1_v, sem1).wait()
            pltpu.async_copy(acc_sh.at[ix0_v], rows0_v, sem0)
            r1 = pl.multiple_of(row0 + c1 * rb, 8)
            pltpu.sync_copy(rows1_v, out_acc.at[cid, pl.ds(r1, rb)])
            return carry

        if nb % 2 == 0:
            lax.fori_loop(0, nb // 2, read_pair, 0)
            pltpu.make_async_copy(acc_sh.at[ix0_v], rows0_v, sem0).wait()
        else:
            lax.fori_loop(0, nb // 2, read_pair, 0)
            pltpu.make_async_copy(acc_sh.at[ix0_v], rows0_v, sem0).wait()
            r = pl.multiple_of(row0 + (nb - 1) * rb, 8)
            pltpu.sync_copy(rows0_v, out_acc.at[cid, pl.ds(r, rb)])
        if with_deg:
            pltpu.sync_copy(deg_v, out_deg.at[wid])

    return sc_agg


_sc_agg_deg = _make_sc_agg(H, True, 80)
_sc_agg_h = _make_sc_agg(H, False, 128)


def kernel(x, edge_index, W_self0, W_neigh0, b0, W_self1, W_neigh1, b1,
           W_self2, W_neigh2, b2):
    src = jnp.concatenate([edge_index[0], jnp.zeros((E2 - E,), jnp.int32)])
    dst = jnp.concatenate(
        [edge_index[1], jnp.full((E2 - E,), NP - 1, jnp.int32)])
    zH80 = jnp.zeros((80, H), jnp.float32)
    z128 = jnp.zeros((128, H), jnp.float32)
    zdeg = jnp.zeros((NP,), jnp.float32)

    t0 = _matmul(x, W_neigh0)
    acc0, degp = _sc_agg_deg(t0, src, dst, zH80, zdeg)
    dp = degp[:, :N]
    h1 = _combine(x, W_self0, b0.reshape(1, H), acc0[0, :N], acc0[1, :N],
                  dp, relu=True)

    t1 = _matmul(h1, W_neigh1)
    acc1 = _sc_agg_h(t1, src, dst, z128)[0]
    h2 = _combine(h1, W_self1, b1.reshape(1, H), acc1[0, :N], acc1[1, :N],
                  dp, relu=True)

    Wn2 = jnp.pad(W_neigh2, ((0, 0), (0, CP - C)))
    Ws2 = jnp.pad(W_self2, ((0, 0), (0, CP - C)))
    b2p = jnp.pad(b2, (0, CP - C)).reshape(1, CP)
    t2 = _matmul(h2, Wn2)
    acc2 = _sc_agg_h(t2, src, dst, z128)[0]
    out = _combine(h2, Ws2, b2p, acc2[0, :N], acc2[1, :N], dp, relu=False)
    return out[:, :C]
